# Initial kernel scaffold; baseline (speedup 1.0000x reference)
import jax
import jax.numpy as jnp
from jax.experimental import pallas as pl


def kernel(x, token_table, added_table, numbers_table, vqgan_codebook, proj_W):
    raise NotImplementedError("write your pallas kernel here")


# speedup vs baseline: 1.6198x; 1.6198x over previous
import functools

import jax
import jax.numpy as jnp
from jax import lax
from jax.experimental import pallas as pl
from jax.experimental.pallas import tpu as pltpu
from jax.experimental.pallas import tpu_sc as plsc

EMBED_DIM = 1024
ADDED_OFFSET = 50000
SINCOS = 1000
VQ_START = 56000
VQ_END = 64192
VQ_VOCAB = 8192
VQ_DIM = 256

NC, NS, L = 2, 16, 16
NW = NC * NS
NTOK = 4 * 4096
BPW = NTOK // NW
K = 32
NCH = BPW // K
FLAT = (NCH + 1) * K
DUMMY = NTOK
OUT_ROWS = NTOK + 8

RANGES = (
    (0, ADDED_OFFSET),
    (ADDED_OFFSET, ADDED_OFFSET + SINCOS),
    (ADDED_OFFSET + SINCOS, VQ_START),
    (VQ_START, VQ_END),
)

_SC_SCRATCH = (
    [pltpu.VMEM((BPW,), jnp.int32)]
    + [pltpu.VMEM((FLAT,), jnp.int32) for _ in range(4)]
    + [pltpu.VMEM((FLAT,), jnp.int32) for _ in range(4)]
    + [pltpu.VMEM((NCH + 1, K), jnp.int32) for _ in range(4)]
    + [pltpu.VMEM((K, EMBED_DIM), jnp.float32), pltpu.SemaphoreType.DMA]
)


@functools.partial(
    pl.kernel,
    out_type=jax.ShapeDtypeStruct((OUT_ROWS, EMBED_DIM), jnp.float32),
    mesh=plsc.VectorSubcoreMesh(core_axis_name="c", subcore_axis_name="s"),
    scratch_types=_SC_SCRATCH,
    compiler_params=pltpu.CompilerParams(needs_layout_passes=False),
)
def _sc_assemble(tok_hbm, num_hbm, add_hbm, img_hbm, x_hbm, out_hbm,
                 ids_v, i0, i1, i2, i3, p0, p1, p2, p3, q0, q1, q2, q3,
                 rows_v, sem):
    tables = (tok_hbm, num_hbm, add_hbm, img_hbm)
    idxs = (i0, i1, i2, i3)
    poss = (p0, p1, p2, p3)
    pos2d = (q0, q1, q2, q3)

    wid = lax.axis_index("s") * NC + lax.axis_index("c")
    base = wid * BPW
    pltpu.sync_copy(x_hbm.at[pl.ds(base, BPW)], ids_v)

    zeros = jnp.zeros((L,), jnp.int32)
    dummy = jnp.full((L,), DUMMY, jnp.int32)

    def init_step(j, carry):
        off = j * L
        for r in range(4):
            idxs[r][pl.ds(off, L)] = zeros
            poss[r][pl.ds(off, L)] = dummy
        return carry

    lax.fori_loop(0, FLAT // L, init_step, 0)

    lane = lax.iota(jnp.int32, L)

    def compact_step(j, cnts):
        v = ids_v[pl.ds(j * L, L)]
        p = base + j * L + lane
        new_cnts = []
        for r, (lo, hi) in enumerate(RANGES):
            m = (v >= lo) & (v < hi)
            mi = m.astype(jnp.int32)
            scan = plsc.cumsum(mi)
            dest = cnts[r] + scan - mi
            plsc.store_scatter(idxs[r], [dest], v - lo, mask=m)
            plsc.store_scatter(poss[r], [dest], p, mask=m)
            new_cnts.append(cnts[r] + scan[L - 1])
        return tuple(new_cnts)

    cnts = lax.fori_loop(
        0, BPW // L, compact_step,
        (jnp.int32(0), jnp.int32(0), jnp.int32(0), jnp.int32(0)))

    def repack_step(j, carry):
        for r in range(4):
            pos2d[r][j, pl.ds(0, L)] = poss[r][pl.ds(j * K, L)]
            pos2d[r][j, pl.ds(L, L)] = poss[r][pl.ds(j * K + L, L)]
        return carry

    lax.fori_loop(0, NCH + 1, repack_step, 0)

    for r in range(4):
        nch = (cnts[r] + (K - 1)) // K

        def chunk_step(c, carry, r=r):
            pltpu.async_copy(
                tables[r].at[idxs[r].at[pl.ds(c * K, K)]], rows_v, sem).wait()
            pltpu.async_copy(rows_v, out_hbm.at[pos2d[r].at[c]], sem).wait()
            return carry

        lax.fori_loop(0, nch, chunk_step, 0)


def _proj_body(cb_ref, w_ref, out_ref):
    out_ref[:] = lax.dot_general(
        cb_ref[:], w_ref[:], (((1,), (1,)), ((), ())),
        preferred_element_type=jnp.float32)


def _project(cb, w):
    return pl.pallas_call(
        _proj_body,
        grid=(8,),
        in_specs=[
            pl.BlockSpec((VQ_VOCAB // 8, VQ_DIM), lambda i: (i, 0)),
            pl.BlockSpec((EMBED_DIM, VQ_DIM), lambda i: (0, 0)),
        ],
        out_specs=pl.BlockSpec((VQ_VOCAB // 8, EMBED_DIM), lambda i: (i, 0)),
        out_shape=jax.ShapeDtypeStruct((VQ_VOCAB, EMBED_DIM), jnp.float32),
    )(cb, w)


def kernel(x, token_table, added_table, numbers_table, vqgan_codebook, proj_W):
    img_table = _project(vqgan_codebook, proj_W)
    xf = x.reshape(-1).astype(jnp.int32)
    out = _sc_assemble(token_table, numbers_table, added_table, img_table, xf)
    return out[:NTOK].reshape(x.shape[0], x.shape[1], EMBED_DIM)


# speedup vs baseline: 3.5398x; 2.1853x over previous
import functools

import jax
import jax.numpy as jnp
from jax import lax
from jax.experimental import pallas as pl
from jax.experimental.pallas import tpu as pltpu
from jax.experimental.pallas import tpu_sc as plsc

EMBED_DIM = 1024
ADDED_OFFSET = 50000
SINCOS = 1000
VQ_START = 56000
VQ_END = 64192
VQ_VOCAB = 8192
VQ_DIM = 256

NC, NS, L = 2, 16, 16
NW = NC * NS
NTOK = 4 * 4096
BPW = NTOK // NW
K = 32
NCH = BPW // K
FLAT = (NCH + 1) * K

RANGES = (
    (0, ADDED_OFFSET),
    (ADDED_OFFSET, ADDED_OFFSET + SINCOS),
    (ADDED_OFFSET + SINCOS, VQ_START),
    (VQ_START, VQ_END),
)

_SC_SCRATCH = (
    [pltpu.VMEM((BPW,), jnp.int32)]
    + [pltpu.VMEM((FLAT,), jnp.int32) for _ in range(4)]
    + [pltpu.VMEM((FLAT,), jnp.int32) for _ in range(4)]
    + [pltpu.VMEM((NCH + 1, K), jnp.int32) for _ in range(4)]
    + [pltpu.VMEM((K, EMBED_DIM), jnp.float32) for _ in range(2)]
    + [pltpu.SemaphoreType.DMA for _ in range(3)]
)


@functools.partial(
    pl.kernel,
    out_type=jax.ShapeDtypeStruct((NTOK, EMBED_DIM), jnp.float32),
    mesh=plsc.VectorSubcoreMesh(core_axis_name="c", subcore_axis_name="s"),
    scratch_types=_SC_SCRATCH,
    compiler_params=pltpu.CompilerParams(needs_layout_passes=False),
)
def _sc_assemble(tok_hbm, num_hbm, add_hbm, img_hbm, x_hbm, out_hbm,
                 ids_v, i0, i1, i2, i3, p0, p1, p2, p3, q0, q1, q2, q3,
                 rows_a, rows_b, gsem, ssem_a, ssem_b):
    tables = (tok_hbm, num_hbm, add_hbm, img_hbm)
    idxs = (i0, i1, i2, i3)
    poss = (p0, p1, p2, p3)
    pos2d = (q0, q1, q2, q3)

    wid = lax.axis_index("s") * NC + lax.axis_index("c")
    base = wid * BPW
    pltpu.sync_copy(x_hbm.at[pl.ds(base, BPW)], ids_v)

    lane = lax.iota(jnp.int32, L)

    def compact_step(j, cnts):
        v = ids_v[pl.ds(j * L, L)]
        p = base + j * L + lane
        new_cnts = []
        for r, (lo, hi) in enumerate(RANGES):
            m = (v >= lo) & (v < hi)
            mi = m.astype(jnp.int32)
            scan = plsc.cumsum(mi)
            dest = cnts[r] + scan - mi
            plsc.store_scatter(idxs[r], [dest], v - lo, mask=m)
            plsc.store_scatter(poss[r], [dest], p, mask=m)
            new_cnts.append(cnts[r] + scan[L - 1])
        return tuple(new_cnts)

    cnts = lax.fori_loop(
        0, BPW // L, compact_step,
        (jnp.int32(0), jnp.int32(0), jnp.int32(0), jnp.int32(0)))

    for r in range(4):
        @pl.when(cnts[r] > 0)
        def _(r=r):
            di = jnp.broadcast_to(idxs[r][pl.ds(0, L)][0], (L,))
            dp = jnp.broadcast_to(poss[r][pl.ds(0, L)][0], (L,))
            idxs[r][pl.ds(cnts[r], L)] = di
            idxs[r][pl.ds(cnts[r] + L, L)] = di
            poss[r][pl.ds(cnts[r], L)] = dp
            poss[r][pl.ds(cnts[r] + L, L)] = dp

    def repack_step(j, carry):
        for r in range(4):
            pos2d[r][j, pl.ds(0, L)] = poss[r][pl.ds(j * K, L)]
            pos2d[r][j, pl.ds(L, L)] = poss[r][pl.ds(j * K + L, L)]
        return carry

    lax.fori_loop(0, NCH + 1, repack_step, 0)

    def wait_scatter(sem, buf):
        pltpu.make_async_copy(out_hbm.at[pl.ds(0, K)], buf, sem).wait()

    for r in range(4):
        nch = (cnts[r] + (K - 1)) // K
        npair = (nch + 1) // 2

        def pair_step(i, carry, r=r, nch=nch):
            c0 = 2 * i

            @pl.when(i > 0)
            def _():
                wait_scatter(ssem_a, rows_a)

            pltpu.async_copy(
                tables[r].at[idxs[r].at[pl.ds(c0 * K, K)]], rows_a, gsem
            ).wait()
            pltpu.async_copy(rows_a, out_hbm.at[pos2d[r].at[c0]], ssem_a)

            @pl.when(c0 + 1 < nch)
            def _():
                @pl.when(i > 0)
                def _():
                    wait_scatter(ssem_b, rows_b)

                pltpu.async_copy(
                    tables[r].at[idxs[r].at[pl.ds((c0 + 1) * K, K)]],
                    rows_b, gsem
                ).wait()
                pltpu.async_copy(
                    rows_b, out_hbm.at[pos2d[r].at[c0 + 1]], ssem_b)

            return carry

        lax.fori_loop(0, npair, pair_step, 0)

        @pl.when(nch > 0)
        def _():
            wait_scatter(ssem_a, rows_a)

        @pl.when((nch > 0) & (nch % 2 == 0))
        def _():
            wait_scatter(ssem_b, rows_b)


def _proj_body(cb_ref, w_ref, out_ref):
    out_ref[:] = lax.dot_general(
        cb_ref[:], w_ref[:], (((1,), (1,)), ((), ())),
        preferred_element_type=jnp.float32)


def _project(cb, w):
    return pl.pallas_call(
        _proj_body,
        grid=(8,),
        in_specs=[
            pl.BlockSpec((VQ_VOCAB // 8, VQ_DIM), lambda i: (i, 0)),
            pl.BlockSpec((EMBED_DIM, VQ_DIM), lambda i: (0, 0)),
        ],
        out_specs=pl.BlockSpec((VQ_VOCAB // 8, EMBED_DIM), lambda i: (i, 0)),
        out_shape=jax.ShapeDtypeStruct((VQ_VOCAB, EMBED_DIM), jnp.float32),
    )(cb, w)


def kernel(x, token_table, added_table, numbers_table, vqgan_codebook, proj_W):
    img_table = _project(vqgan_codebook, proj_W)
    xf = x.reshape(-1).astype(jnp.int32)
    out = _sc_assemble(token_table, numbers_table, added_table, img_table, xf)
    return out.reshape(x.shape[0], x.shape[1], EMBED_DIM)


# speedup vs baseline: 3.8009x; 1.0738x over previous
import functools

import jax
import jax.numpy as jnp
from jax import lax
from jax.experimental import pallas as pl
from jax.experimental.pallas import tpu as pltpu
from jax.experimental.pallas import tpu_sc as plsc

EMBED_DIM = 1024
ADDED_OFFSET = 50000
SINCOS = 1000
VQ_START = 56000
VQ_END = 64192
VQ_VOCAB = 8192
VQ_DIM = 256

NC, NS, L = 2, 16, 16
NW = NC * NS
NTOK = 4 * 4096
BPW = NTOK // NW
K = 32
NCH = BPW // K
FLAT = (NCH + 1) * K

RANGES = (
    (0, ADDED_OFFSET),
    (ADDED_OFFSET, ADDED_OFFSET + SINCOS),
    (ADDED_OFFSET + SINCOS, VQ_START),
    (VQ_START, VQ_END),
)

_SC_SCRATCH = (
    [pltpu.VMEM((BPW,), jnp.int32)]
    + [pltpu.VMEM((FLAT,), jnp.int32) for _ in range(4)]
    + [pltpu.VMEM((FLAT,), jnp.int32) for _ in range(4)]
    + [pltpu.VMEM((NCH + 1, K), jnp.int32) for _ in range(4)]
    + [pltpu.VMEM((K, EMBED_DIM), jnp.float32) for _ in range(3)]
    + [pltpu.SemaphoreType.DMA for _ in range(6)]
)
NBUF = 3


@functools.partial(
    pl.kernel,
    out_type=jax.ShapeDtypeStruct((NTOK, EMBED_DIM), jnp.float32),
    mesh=plsc.VectorSubcoreMesh(core_axis_name="c", subcore_axis_name="s"),
    scratch_types=_SC_SCRATCH,
    compiler_params=pltpu.CompilerParams(needs_layout_passes=False),
)
def _sc_assemble(tok_hbm, num_hbm, add_hbm, img_hbm, x_hbm, out_hbm,
                 ids_v, i0, i1, i2, i3, p0, p1, p2, p3, q0, q1, q2, q3,
                 rows_a, rows_b, rows_c, gsem_a, gsem_b, gsem_c,
                 ssem_a, ssem_b, ssem_c):
    rows = (rows_a, rows_b, rows_c)
    gsems = (gsem_a, gsem_b, gsem_c)
    ssems = (ssem_a, ssem_b, ssem_c)
    tables = (tok_hbm, num_hbm, add_hbm, img_hbm)
    idxs = (i0, i1, i2, i3)
    poss = (p0, p1, p2, p3)
    pos2d = (q0, q1, q2, q3)

    wid = lax.axis_index("s") * NC + lax.axis_index("c")
    base = wid * BPW
    pltpu.sync_copy(x_hbm.at[pl.ds(base, BPW)], ids_v)

    lane = lax.iota(jnp.int32, L)

    def compact_step(j, cnts):
        v = ids_v[pl.ds(j * L, L)]
        p = base + j * L + lane
        new_cnts = []
        for r, (lo, hi) in enumerate(RANGES):
            m = (v >= lo) & (v < hi)
            mi = m.astype(jnp.int32)
            scan = plsc.cumsum(mi)
            dest = cnts[r] + scan - mi
            plsc.store_scatter(idxs[r], [dest], v - lo, mask=m)
            plsc.store_scatter(poss[r], [dest], p, mask=m)
            new_cnts.append(cnts[r] + scan[L - 1])
        return tuple(new_cnts)

    cnts = lax.fori_loop(
        0, BPW // L, compact_step,
        (jnp.int32(0), jnp.int32(0), jnp.int32(0), jnp.int32(0)))

    for r in range(4):
        @pl.when(cnts[r] > 0)
        def _(r=r):
            di = jnp.broadcast_to(idxs[r][pl.ds(0, L)][0], (L,))
            dp = jnp.broadcast_to(poss[r][pl.ds(0, L)][0], (L,))
            idxs[r][pl.ds(cnts[r], L)] = di
            idxs[r][pl.ds(cnts[r] + L, L)] = di
            poss[r][pl.ds(cnts[r], L)] = dp
            poss[r][pl.ds(cnts[r] + L, L)] = dp

    def repack_step(j, carry):
        for r in range(4):
            pos2d[r][j, pl.ds(0, L)] = poss[r][pl.ds(j * K, L)]
            pos2d[r][j, pl.ds(L, L)] = poss[r][pl.ds(j * K + L, L)]
        return carry

    lax.fori_loop(0, NCH + 1, repack_step, 0)

    def wait_sem(sem, buf):
        pltpu.make_async_copy(out_hbm.at[pl.ds(0, K)], buf, sem).wait()

    for r in range(4):
        nch = (cnts[r] + (K - 1)) // K
        ngrp = (nch + (NBUF - 1)) // NBUF

        def grp_step(i, carry, r=r, nch=nch):
            c0 = NBUF * i
            for s in range(NBUF):
                @pl.when(c0 + s < nch)
                def _(s=s):
                    @pl.when(i > 0)
                    def _():
                        wait_sem(ssems[s], rows[s])

                    pltpu.async_copy(
                        tables[r].at[idxs[r].at[pl.ds((c0 + s) * K, K)]],
                        rows[s], gsems[s])

            for s in range(NBUF):
                @pl.when(c0 + s < nch)
                def _(s=s):
                    wait_sem(gsems[s], rows[s])
                    pltpu.async_copy(
                        rows[s], out_hbm.at[pos2d[r].at[c0 + s]], ssems[s])

            return carry

        lax.fori_loop(0, ngrp, grp_step, 0)

        for s in range(NBUF):
            @pl.when(nch > s)
            def _(s=s):
                wait_sem(ssems[s], rows[s])


def _proj_body(cb_ref, w_ref, out_ref):
    out_ref[:] = lax.dot_general(
        cb_ref[:], w_ref[:], (((1,), (1,)), ((), ())),
        preferred_element_type=jnp.float32)


def _project(cb, w):
    return pl.pallas_call(
        _proj_body,
        grid=(8,),
        in_specs=[
            pl.BlockSpec((VQ_VOCAB // 8, VQ_DIM), lambda i: (i, 0)),
            pl.BlockSpec((EMBED_DIM, VQ_DIM), lambda i: (0, 0)),
        ],
        out_specs=pl.BlockSpec((VQ_VOCAB // 8, EMBED_DIM), lambda i: (i, 0)),
        out_shape=jax.ShapeDtypeStruct((VQ_VOCAB, EMBED_DIM), jnp.float32),
    )(cb, w)


def kernel(x, token_table, added_table, numbers_table, vqgan_codebook, proj_W):
    img_table = _project(vqgan_codebook, proj_W)
    xf = x.reshape(-1).astype(jnp.int32)
    out = _sc_assemble(token_table, numbers_table, added_table, img_table, xf)
    return out.reshape(x.shape[0], x.shape[1], EMBED_DIM)
